# baseline (speedup 1.0000x reference)
import functools

import jax
import jax.numpy as jnp
from jax import lax
from jax.experimental import pallas as pl
from jax.experimental.pallas import tpu as pltpu
from jax.experimental.pallas import tpu_sc as plsc

_ROWS_PER_WORKER = 8


@functools.lru_cache(maxsize=None)
def _make_sc_gather(num_rows, d_model, max_len):
    info = plsc.get_sparse_core_info()
    num_cores = info.num_cores
    mesh = plsc.VectorSubcoreMesh(core_axis_name="c", subcore_axis_name="s")
    active = num_rows // _ROWS_PER_WORKER

    @functools.partial(
        pl.kernel,
        mesh=mesh,
        out_type=jax.ShapeDtypeStruct((num_rows, d_model), jnp.float32),
        scratch_types=[
            pltpu.VMEM((_ROWS_PER_WORKER,), jnp.int32),
            pltpu.VMEM((_ROWS_PER_WORKER, d_model), jnp.float32),
            pltpu.SemaphoreType.DMA,
        ],
    )
    def gather(pe_hbm, idx_hbm, out_hbm, idx_v, rows_v, sem):
        wid = lax.axis_index("s") * num_cores + lax.axis_index("c")

        @pl.when(wid < active)
        def _():
            base = wid * _ROWS_PER_WORKER
            pltpu.sync_copy(idx_hbm.at[pl.ds(base, _ROWS_PER_WORKER)], idx_v)
            pltpu.async_copy(pe_hbm.at[idx_v], rows_v, sem).wait()
            pltpu.sync_copy(rows_v, out_hbm.at[pl.ds(base, _ROWS_PER_WORKER)])

    return gather


def _make_add_body(G):
    def _add_body(x_ref, t_ref, o_ref):
        for k in range(G):
            o_ref[k] = x_ref[k] + t_ref[k : k + 1, :]

    return _add_body


def kernel(x, batch_positions, pe):
    B, T, d_model, H, W = x.shape
    BT = B * T
    HW = H * W
    x3 = x.transpose(0, 1, 3, 4, 2).reshape(BT, HW, d_model)
    pos = batch_positions.reshape(BT)

    table = _make_sc_gather(BT, d_model, pe.shape[0])(pe, pos)

    G = 16
    out = pl.pallas_call(
        _make_add_body(G),
        grid=(BT // G,),
        in_specs=[
            pl.BlockSpec((G, HW, d_model), lambda i: (i, 0, 0)),
            pl.BlockSpec((G, d_model), lambda i: (i, 0)),
        ],
        out_specs=pl.BlockSpec((G, HW, d_model), lambda i: (i, 0, 0)),
        out_shape=jax.ShapeDtypeStruct((BT, HW, d_model), jnp.float32),
    )(x3, table)
    return out.reshape(B, T, H, W, d_model).transpose(0, 1, 4, 2, 3)


# speedup vs baseline: 1.0202x; 1.0202x over previous
import functools

import jax
import jax.numpy as jnp
from jax import lax
from jax.experimental import pallas as pl
from jax.experimental.pallas import tpu as pltpu
from jax.experimental.pallas import tpu_sc as plsc

_ROWS_PER_WORKER = 8
_G = 16


@functools.lru_cache(maxsize=None)
def _make_sc_gather(start, num_rows, d_model, max_len):
    info = plsc.get_sparse_core_info()
    num_cores = info.num_cores
    mesh = plsc.VectorSubcoreMesh(core_axis_name="c", subcore_axis_name="s")
    active = num_rows // _ROWS_PER_WORKER

    @functools.partial(
        pl.kernel,
        mesh=mesh,
        out_type=jax.ShapeDtypeStruct((num_rows, d_model), jnp.float32),
        scratch_types=[
            pltpu.VMEM((_ROWS_PER_WORKER,), jnp.int32),
            pltpu.VMEM((_ROWS_PER_WORKER, d_model), jnp.float32),
            pltpu.SemaphoreType.DMA,
        ],
    )
    def gather(pe_hbm, idx_hbm, out_hbm, idx_v, rows_v, sem):
        wid = lax.axis_index("s") * num_cores + lax.axis_index("c")

        @pl.when(wid < active)
        def _():
            base = wid * _ROWS_PER_WORKER
            pltpu.sync_copy(idx_hbm.at[pl.ds(start + base, _ROWS_PER_WORKER)], idx_v)
            pltpu.async_copy(pe_hbm.at[idx_v], rows_v, sem).wait()
            pltpu.sync_copy(rows_v, out_hbm.at[pl.ds(base, _ROWS_PER_WORKER)])

    return gather


def _add_body1(pos_ref, x_ref, *pe_and_out):
    pe_refs, o_ref = pe_and_out[:-1], pe_and_out[-1]
    i = pl.program_id(0)
    for k in range(_G):
        s = pos_ref[i * _G + k] % _ROWS_PER_WORKER
        o_ref[k] = x_ref[k] + pe_refs[k][pl.ds(s, 1), :]


def _add_body2(x_ref, t_ref, prev_ref, o_ref):
    del prev_ref
    for k in range(_G):
        o_ref[k] = x_ref[k] + t_ref[k : k + 1, :]


def kernel(x, batch_positions, pe):
    B, T, d_model, H, W = x.shape
    BT = B * T
    HW = H * W
    half = BT // 2
    n1 = half // _G
    x3 = x.transpose(0, 1, 3, 4, 2).reshape(BT, HW, d_model)
    pos = batch_positions.reshape(BT)

    table2 = _make_sc_gather(half, half, d_model, pe.shape[0])(pe, pos)

    def _pe_map(k):
        return lambda i, pref: (pref[i * _G + k] // _ROWS_PER_WORKER, 0)

    grid_spec = pltpu.PrefetchScalarGridSpec(
        num_scalar_prefetch=1,
        grid=(n1,),
        in_specs=[
            pl.BlockSpec((_G, HW, d_model), lambda i, pref: (i, 0, 0)),
            *[
                pl.BlockSpec((_ROWS_PER_WORKER, d_model), _pe_map(k))
                for k in range(_G)
            ],
        ],
        out_specs=pl.BlockSpec((_G, HW, d_model), lambda i, pref: (i, 0, 0)),
    )
    out1 = pl.pallas_call(
        _add_body1,
        grid_spec=grid_spec,
        out_shape=jax.ShapeDtypeStruct((BT, HW, d_model), jnp.float32),
    )(pos, x3, *([pe] * _G))

    out = pl.pallas_call(
        _add_body2,
        grid=(n1,),
        in_specs=[
            pl.BlockSpec((_G, HW, d_model), lambda i: (i + n1, 0, 0)),
            pl.BlockSpec((_G, d_model), lambda i: (i, 0)),
            pl.BlockSpec(memory_space=pltpu.MemorySpace.HBM),
        ],
        out_specs=pl.BlockSpec((_G, HW, d_model), lambda i: (i + n1, 0, 0)),
        out_shape=jax.ShapeDtypeStruct((BT, HW, d_model), jnp.float32),
        input_output_aliases={2: 0},
    )(x3, table2, out1)
    return out.reshape(B, T, H, W, d_model).transpose(0, 1, 4, 2, 3)


# speedup vs baseline: 1.0269x; 1.0066x over previous
import functools

import jax
import jax.numpy as jnp
from jax import lax
from jax.experimental import pallas as pl
from jax.experimental.pallas import tpu as pltpu
from jax.experimental.pallas import tpu_sc as plsc

_ROWS_PER_WORKER = 8
_G = 16


@functools.lru_cache(maxsize=None)
def _make_sc_gather(start, num_rows, d_model, max_len):
    info = plsc.get_sparse_core_info()
    num_cores = info.num_cores
    mesh = plsc.VectorSubcoreMesh(core_axis_name="c", subcore_axis_name="s")
    active = num_rows // _ROWS_PER_WORKER

    @functools.partial(
        pl.kernel,
        mesh=mesh,
        out_type=jax.ShapeDtypeStruct((num_rows, d_model), jnp.float32),
        scratch_types=[
            pltpu.VMEM((_ROWS_PER_WORKER,), jnp.int32),
            pltpu.VMEM((_ROWS_PER_WORKER, d_model), jnp.float32),
            pltpu.SemaphoreType.DMA,
        ],
    )
    def gather(pe_hbm, idx_hbm, out_hbm, idx_v, rows_v, sem):
        wid = lax.axis_index("s") * num_cores + lax.axis_index("c")
        T = idx_hbm.shape[1]

        @pl.when(wid < active)
        def _():
            base = wid * _ROWS_PER_WORKER
            off = start + base
            pltpu.sync_copy(
                idx_hbm.at[off // T, pl.ds(off % T, _ROWS_PER_WORKER)], idx_v
            )
            pltpu.async_copy(pe_hbm.at[idx_v], rows_v, sem).wait()
            pltpu.sync_copy(rows_v, out_hbm.at[pl.ds(base, _ROWS_PER_WORKER)])

    return gather


def _add_body1(pos_ref, x_ref, *pe_and_out):
    pe_refs, o_ref = pe_and_out[:-1], pe_and_out[-1]
    T = pos_ref.shape[1]
    i = pl.program_id(0)
    for k in range(_G):
        bt = i * _G + k
        s = pos_ref[bt // T, bt % T] % _ROWS_PER_WORKER
        o_ref[k] = x_ref[k] + pe_refs[k][pl.ds(s, 1), :]


def _add_body2(x_ref, t_ref, prev_ref, o_ref):
    del prev_ref
    for k in range(_G):
        o_ref[k] = x_ref[k] + t_ref[k : k + 1, :]


def kernel(x, batch_positions, pe):
    B, T, d_model, H, W = x.shape
    BT = B * T
    HW = H * W
    part1 = BT // 3
    part2 = BT - part1
    n1 = part1 // _G
    n2 = part2 // _G
    x3 = x.transpose(0, 1, 3, 4, 2).reshape(BT, HW, d_model)

    table2 = _make_sc_gather(part1, part2, d_model, pe.shape[0])(
        pe, batch_positions
    )

    def _pe_map(k):
        return lambda i, pref: (
            pref[(i * _G + k) // T, (i * _G + k) % T] // _ROWS_PER_WORKER,
            0,
        )

    grid_spec = pltpu.PrefetchScalarGridSpec(
        num_scalar_prefetch=1,
        grid=(n1,),
        in_specs=[
            pl.BlockSpec((_G, HW, d_model), lambda i, pref: (i, 0, 0)),
            *[
                pl.BlockSpec((_ROWS_PER_WORKER, d_model), _pe_map(k))
                for k in range(_G)
            ],
        ],
        out_specs=pl.BlockSpec((_G, HW, d_model), lambda i, pref: (i, 0, 0)),
    )
    out1 = pl.pallas_call(
        _add_body1,
        grid_spec=grid_spec,
        out_shape=jax.ShapeDtypeStruct((BT, HW, d_model), jnp.float32),
    )(batch_positions, x3, *([pe] * _G))

    out = pl.pallas_call(
        _add_body2,
        grid=(n2,),
        in_specs=[
            pl.BlockSpec((_G, HW, d_model), lambda i: (i + n1, 0, 0)),
            pl.BlockSpec((_G, d_model), lambda i: (i, 0)),
            pl.BlockSpec(memory_space=pltpu.MemorySpace.HBM),
        ],
        out_specs=pl.BlockSpec((_G, HW, d_model), lambda i: (i + n1, 0, 0)),
        out_shape=jax.ShapeDtypeStruct((BT, HW, d_model), jnp.float32),
        input_output_aliases={2: 0},
    )(x3, table2, out1)
    return out.reshape(B, T, H, W, d_model).transpose(0, 1, 4, 2, 3)


# speedup vs baseline: 1.0286x; 1.0016x over previous
import functools

import jax
import jax.numpy as jnp
from jax import lax
from jax.experimental import pallas as pl
from jax.experimental.pallas import tpu as pltpu
from jax.experimental.pallas import tpu_sc as plsc

_ROWS_PER_WORKER = 8
_G = 16


@functools.lru_cache(maxsize=None)
def _make_sc_gather(start, num_rows, d_model, max_len):
    info = plsc.get_sparse_core_info()
    num_cores = info.num_cores
    mesh = plsc.VectorSubcoreMesh(core_axis_name="c", subcore_axis_name="s")
    active = num_rows // _ROWS_PER_WORKER

    @functools.partial(
        pl.kernel,
        mesh=mesh,
        out_type=jax.ShapeDtypeStruct((num_rows, d_model), jnp.float32),
        scratch_types=[
            pltpu.VMEM((_ROWS_PER_WORKER,), jnp.int32),
            pltpu.VMEM((_ROWS_PER_WORKER, d_model), jnp.float32),
            pltpu.SemaphoreType.DMA,
        ],
    )
    def gather(pe_hbm, idx_hbm, out_hbm, idx_v, rows_v, sem):
        wid = lax.axis_index("s") * num_cores + lax.axis_index("c")
        T = idx_hbm.shape[1]

        @pl.when(wid < active)
        def _():
            base = wid * _ROWS_PER_WORKER
            off = start + base
            pltpu.sync_copy(
                idx_hbm.at[off // T, pl.ds(off % T, _ROWS_PER_WORKER)], idx_v
            )
            pltpu.async_copy(pe_hbm.at[idx_v], rows_v, sem).wait()
            pltpu.sync_copy(rows_v, out_hbm.at[pl.ds(base, _ROWS_PER_WORKER)])

    return gather


def _add_body1(pos_ref, x_ref, *pe_and_out):
    pe_refs, o_ref = pe_and_out[:-1], pe_and_out[-1]
    T = pos_ref.shape[1]
    i = pl.program_id(0)
    for k in range(_G):
        bt = i * _G + k
        s = pos_ref[bt // T, bt % T] % _ROWS_PER_WORKER
        o_ref[k] = x_ref[k] + pe_refs[k][pl.ds(s, 1), :]


def _add_body2(x_ref, t_ref, prev_ref, o_ref):
    del prev_ref
    for k in range(_G):
        o_ref[k] = x_ref[k] + t_ref[k : k + 1, :]


def kernel(x, batch_positions, pe):
    B, T, d_model, H, W = x.shape
    BT = B * T
    HW = H * W
    part1 = BT // 6
    part2 = BT - part1
    n1 = part1 // _G
    n2 = part2 // _G
    x3 = x.transpose(0, 1, 3, 4, 2).reshape(BT, HW, d_model)

    table2 = _make_sc_gather(part1, part2, d_model, pe.shape[0])(
        pe, batch_positions
    )

    def _pe_map(k):
        return lambda i, pref: (
            pref[(i * _G + k) // T, (i * _G + k) % T] // _ROWS_PER_WORKER,
            0,
        )

    grid_spec = pltpu.PrefetchScalarGridSpec(
        num_scalar_prefetch=1,
        grid=(n1,),
        in_specs=[
            pl.BlockSpec((_G, HW, d_model), lambda i, pref: (i, 0, 0)),
            *[
                pl.BlockSpec((_ROWS_PER_WORKER, d_model), _pe_map(k))
                for k in range(_G)
            ],
        ],
        out_specs=pl.BlockSpec((_G, HW, d_model), lambda i, pref: (i, 0, 0)),
    )
    out1 = pl.pallas_call(
        _add_body1,
        grid_spec=grid_spec,
        out_shape=jax.ShapeDtypeStruct((BT, HW, d_model), jnp.float32),
    )(batch_positions, x3, *([pe] * _G))

    out = pl.pallas_call(
        _add_body2,
        grid=(n2,),
        in_specs=[
            pl.BlockSpec((_G, HW, d_model), lambda i: (i + n1, 0, 0)),
            pl.BlockSpec((_G, d_model), lambda i: (i, 0)),
            pl.BlockSpec(memory_space=pltpu.MemorySpace.HBM),
        ],
        out_specs=pl.BlockSpec((_G, HW, d_model), lambda i: (i + n1, 0, 0)),
        out_shape=jax.ShapeDtypeStruct((BT, HW, d_model), jnp.float32),
        input_output_aliases={2: 0},
    )(x3, table2, out1)
    return out.reshape(B, T, H, W, d_model).transpose(0, 1, 4, 2, 3)


# speedup vs baseline: 1.0457x; 1.0167x over previous
import functools

import jax
import jax.numpy as jnp
from jax import lax
from jax.experimental import pallas as pl
from jax.experimental.pallas import tpu as pltpu
from jax.experimental.pallas import tpu_sc as plsc

_ROWS_PER_WORKER = 8
_G = 16


@functools.lru_cache(maxsize=None)
def _make_sc_gather(start, num_rows, d_model, max_len):
    info = plsc.get_sparse_core_info()
    num_cores = info.num_cores
    mesh = plsc.VectorSubcoreMesh(core_axis_name="c", subcore_axis_name="s", num_cores=1)
    active = num_rows // _ROWS_PER_WORKER

    @functools.partial(
        pl.kernel,
        mesh=mesh,
        out_type=jax.ShapeDtypeStruct((num_rows, d_model), jnp.float32),
        scratch_types=[
            pltpu.VMEM((_ROWS_PER_WORKER,), jnp.int32),
            pltpu.VMEM((_ROWS_PER_WORKER, d_model), jnp.float32),
            pltpu.SemaphoreType.DMA,
        ],
    )
    def gather(pe_hbm, idx_hbm, out_hbm, idx_v, rows_v, sem):
        wid = lax.axis_index("s") * num_cores + lax.axis_index("c")
        T = idx_hbm.shape[1]

        @pl.when(wid < active)
        def _():
            base = wid * _ROWS_PER_WORKER
            off = start + base
            pltpu.sync_copy(
                idx_hbm.at[off // T, pl.ds(off % T, _ROWS_PER_WORKER)], idx_v
            )
            pltpu.async_copy(pe_hbm.at[idx_v], rows_v, sem).wait()
            pltpu.sync_copy(rows_v, out_hbm.at[pl.ds(base, _ROWS_PER_WORKER)])

    return gather


def _add_body1(pos_ref, x_ref, *pe_and_out):
    pe_refs, o_ref = pe_and_out[:-1], pe_and_out[-1]
    T = pos_ref.shape[1]
    i = pl.program_id(0)
    for k in range(_G):
        bt = i * _G + k
        s = pos_ref[bt // T, bt % T] % _ROWS_PER_WORKER
        o_ref[k] = x_ref[k] + pe_refs[k][pl.ds(s, 1), :]


def _add_body2(x_ref, t_ref, prev_ref, o_ref):
    del prev_ref
    for k in range(_G):
        o_ref[k] = x_ref[k] + t_ref[k : k + 1, :]


def kernel(x, batch_positions, pe):
    B, T, d_model, H, W = x.shape
    BT = B * T
    HW = H * W
    part1 = BT // 3
    part2 = BT - part1
    n1 = part1 // _G
    n2 = part2 // _G
    x3 = x.transpose(0, 1, 3, 4, 2).reshape(BT, HW, d_model)

    table2 = _make_sc_gather(part1, part2, d_model, pe.shape[0])(
        pe, batch_positions
    )

    def _pe_map(k):
        return lambda i, pref: (
            pref[(i * _G + k) // T, (i * _G + k) % T] // _ROWS_PER_WORKER,
            0,
        )

    grid_spec = pltpu.PrefetchScalarGridSpec(
        num_scalar_prefetch=1,
        grid=(n1,),
        in_specs=[
            pl.BlockSpec((_G, HW, d_model), lambda i, pref: (i, 0, 0)),
            *[
                pl.BlockSpec((_ROWS_PER_WORKER, d_model), _pe_map(k))
                for k in range(_G)
            ],
        ],
        out_specs=pl.BlockSpec((_G, HW, d_model), lambda i, pref: (i, 0, 0)),
    )
    out1 = pl.pallas_call(
        _add_body1,
        grid_spec=grid_spec,
        out_shape=jax.ShapeDtypeStruct((BT, HW, d_model), jnp.float32),
    )(batch_positions, x3, *([pe] * _G))

    out = pl.pallas_call(
        _add_body2,
        grid=(n2,),
        in_specs=[
            pl.BlockSpec((_G, HW, d_model), lambda i: (i + n1, 0, 0)),
            pl.BlockSpec((_G, d_model), lambda i: (i, 0)),
            pl.BlockSpec(memory_space=pltpu.MemorySpace.HBM),
        ],
        out_specs=pl.BlockSpec((_G, HW, d_model), lambda i: (i + n1, 0, 0)),
        out_shape=jax.ShapeDtypeStruct((BT, HW, d_model), jnp.float32),
        input_output_aliases={2: 0},
    )(x3, table2, out1)
    return out.reshape(B, T, H, W, d_model).transpose(0, 1, 4, 2, 3)
